# baseline (speedup 1.0000x reference)
import jax
import jax.numpy as jnp
from jax import lax
from jax.experimental import pallas as pl
from jax.experimental.pallas import tpu as pltpu
from jax.experimental.pallas import tpu_sc as plsc

_VOC = 1000000
_VMAIN = 999936
_BATCH = 4096
_TOK = 200
_D = 64
_NW = 32
_BPW = _BATCH // _NW
_CV = 128
_CPW = _VMAIN // (_NW * _CV)
_STRIPE = _CPW * _CV
_TAIL = _NW * _STRIPE
_SCALE = 8.0


def _tr_body(tabT, out_hbm, i0, i1, o0, o1, is0, is1, os0, os1):
    c = lax.axis_index("c")
    s = lax.axis_index("s")
    wid = s * 2 + c
    base = wid * _STRIPE
    iota = lax.iota(jnp.int32, 16)

    ibuf = (i0, i1)
    obuf = (o0, o1)
    isem = (is0, is1)
    osem = (os0, os1)

    def start_in(v0, b):
        pltpu.async_copy(tabT.at[:, pl.ds(v0, _CV)], ibuf[b], isem[b])

    def wait_in(v0, b):
        pltpu.make_async_copy(tabT.at[:, pl.ds(v0, _CV)], ibuf[b],
                              isem[b]).wait()

    def start_out(v0, b):
        pltpu.async_copy(obuf[b], out_hbm.at[pl.ds(v0 * _D, _CV * _D)],
                         osem[b])

    def wait_out(v0, b):
        pltpu.make_async_copy(obuf[b],
                              out_hbm.at[pl.ds(v0 * _D, _CV * _D)],
                              osem[b]).wait()

    def compute(b):
        def grp(vg, carry):
            fbase = (vg * 16 + iota) * _D
            for col in range(_D):
                plsc.store_scatter(obuf[b], [fbase + col],
                                   ibuf[b][col, pl.ds(vg * 16, 16)])
            return carry
        lax.fori_loop(0, _CV // 16, grp, 0)

    start_in(base, 0)
    start_in(base + _CV, 1)
    for k in (0, 1):
        b = k & 1
        wait_in(base + k * _CV, b)
        compute(b)
        start_out(base + k * _CV, b)
        start_in(base + (k + 2) * _CV, b)

    def steady(ko, carry):
        for b in range(2):
            v0 = base + (ko * 2 + b) * _CV
            wait_out(v0 - 2 * _CV, b)
            wait_in(v0, b)
            compute(b)
            start_out(v0, b)
            start_in(v0 + 2 * _CV, b)
        return carry

    lax.fori_loop(1, _CPW // 2 - 1, steady, 0)

    for k in (_CPW - 2, _CPW - 1):
        b = k & 1
        v0 = base + k * _CV
        wait_out(v0 - 2 * _CV, b)
        wait_in(v0, b)
        compute(b)
        start_out(v0, b)
    for k in (_CPW - 2, _CPW - 1):
        wait_out(base + k * _CV, k & 1)

    @pl.when(wid < 4)
    def _():
        v0 = _TAIL + wid * _CV
        pltpu.sync_copy(tabT.at[:, pl.ds(v0, _CV)], ibuf[0])
        compute(0)
        pltpu.sync_copy(obuf[0], out_hbm.at[pl.ds(v0 * _D, _CV * _D)])


def _emb_body(idxT, tab_hbm, tail_hbm, out_hbm, idx_v, tail_v,
              g0, g1, o0, o1, gs0, gs1, os0, os1):
    c = lax.axis_index("c")
    s = lax.axis_index("s")
    wid = s * 2 + c
    b0 = wid * _BPW
    iota = lax.iota(jnp.int32, 16)

    gbuf = (g0, g1)
    obuf = (o0, o1)
    gsem = (gs0, gs1)
    osem = (os0, os1)

    pltpu.sync_copy(idxT.at[:, pl.ds(b0, _BPW)], idx_v)
    pltpu.sync_copy(tail_hbm, tail_v)

    def start_gather(t, b):
        pltpu.async_copy(tab_hbm.at[idx_v.at[t]], gbuf[b], gsem[b])

    def wait_gather(t, b):
        pltpu.make_async_copy(tab_hbm.at[idx_v.at[t]], gbuf[b],
                              gsem[b]).wait()

    def start_out(t, b):
        pltpu.async_copy(obuf[b], out_hbm.at[t, :, wid, :, :], osem[b])

    def wait_out(t, b):
        pltpu.make_async_copy(obuf[b], out_hbm.at[t, :, wid, :, :],
                              osem[b]).wait()

    def compute(t, b):
        def grp(bg, carry):
            iv = idx_v[t, pl.ds(bg * 16, 16)]
            sc = jnp.where(iv != 0, jnp.float32(_SCALE), jnp.float32(0.0))
            ridx = bg * 16 + iota
            tmask = iv >= _VMAIN

            def plain(col):
                cidx = jnp.full((16,), col, jnp.int32)
                return plsc.load_gather(gbuf[b], [ridx, cidx])

            @pl.when(jnp.logical_not(tmask.any()))
            def _():
                for col in range(_D):
                    obuf[b][col // 8, col % 8, pl.ds(bg * 16, 16)] = (
                        plain(col) * sc)

            @pl.when(tmask.any())
            def _():
                tridx = iv - _VMAIN
                for col in range(_D):
                    cidx = jnp.full((16,), col, jnp.int32)
                    tvals = plsc.load_gather(
                        tail_v, [jnp.where(tmask, tridx, 0), cidx])
                    vals = jnp.where(tmask, tvals, plain(col))
                    obuf[b][col // 8, col % 8, pl.ds(bg * 16, 16)] = vals * sc
            return carry
        lax.fori_loop(0, _BPW // 16, grp, 0)

    start_gather(0, 0)
    start_gather(1, 1)
    for t in (0, 1):
        b = t & 1
        wait_gather(t, b)
        compute(t, b)
        start_out(t, b)
        start_gather(t + 2, b)

    def steady(to, carry):
        for b in range(2):
            t = to * 2 + b
            wait_out(t - 2, b)
            wait_gather(t, b)
            compute(t, b)
            start_out(t, b)
            start_gather(t + 2, b)
        return carry

    lax.fori_loop(1, _TOK // 2 - 1, steady, 0)

    for t in (_TOK - 2, _TOK - 1):
        b = t & 1
        wait_out(t - 2, b)
        wait_gather(t, b)
        compute(t, b)
        start_out(t, b)
    for t in (_TOK - 2, _TOK - 1):
        wait_out(t, t & 1)


def kernel(input, lookup_table):
    idxT = input.astype(jnp.int32).T
    tabT = lookup_table.T
    tab_tail = lookup_table[_VMAIN:, :]
    mesh = plsc.VectorSubcoreMesh(core_axis_name="c", subcore_axis_name="s")

    tab_flat = pl.kernel(
        _tr_body,
        out_type=jax.ShapeDtypeStruct((_VOC * _D,), jnp.float32),
        mesh=mesh,
        compiler_params=pltpu.CompilerParams(
            use_tc_tiling_on_sc=True, needs_layout_passes=False),
        scratch_types=[
            pltpu.VMEM((_D, _CV), jnp.float32),
            pltpu.VMEM((_D, _CV), jnp.float32),
            pltpu.VMEM((_CV * _D,), jnp.float32),
            pltpu.VMEM((_CV * _D,), jnp.float32),
            pltpu.SemaphoreType.DMA,
            pltpu.SemaphoreType.DMA,
            pltpu.SemaphoreType.DMA,
            pltpu.SemaphoreType.DMA,
        ],
    )(tabT)
    tab_lin = tab_flat.reshape(_VOC, _D)

    out5d = pl.kernel(
        _emb_body,
        out_type=jax.ShapeDtypeStruct((_TOK, _D // 8, _BATCH // 128, 8, 128),
                                      jnp.float32),
        mesh=mesh,
        compiler_params=pltpu.CompilerParams(
            use_tc_tiling_on_sc=False, needs_layout_passes=False),
        scratch_types=[
            pltpu.VMEM((_TOK, _BPW), jnp.int32),
            pltpu.VMEM((_D, _D), jnp.float32),
            pltpu.VMEM((_BPW, _D), jnp.float32),
            pltpu.VMEM((_BPW, _D), jnp.float32),
            pltpu.VMEM((_D // 8, 8, _BPW), jnp.float32),
            pltpu.VMEM((_D // 8, 8, _BPW), jnp.float32),
            pltpu.SemaphoreType.DMA,
            pltpu.SemaphoreType.DMA,
            pltpu.SemaphoreType.DMA,
            pltpu.SemaphoreType.DMA,
        ],
    )(idxT, tab_lin, tab_tail)

    return out5d.transpose(2, 4, 0, 1, 3).reshape(_BATCH, _TOK, _D)


# speedup vs baseline: 2.0520x; 2.0520x over previous
import jax
import jax.numpy as jnp
from jax import lax
from jax.experimental import pallas as pl
from jax.experimental.pallas import tpu as pltpu
from jax.experimental.pallas import tpu_sc as plsc

_VOC = 1000000
_VMAIN = 999936
_BATCH = 4096
_TOK = 200
_D = 64
_NW = 32
_BPW = _BATCH // _NW
_CV = 128
_CPW = _VMAIN // (_NW * _CV)
_STRIPE = _CPW * _CV
_TAIL = _NW * _STRIPE
_SCALE = 8.0


def _tr_body(tabT, out_hbm, i0, i1, o0, o1, is0, is1, os0, os1):
    c = lax.axis_index("c")
    s = lax.axis_index("s")
    wid = s * 2 + c
    base = wid * _STRIPE
    iota = lax.iota(jnp.int32, 16)

    ibuf = (i0, i1)
    obuf = (o0, o1)
    isem = (is0, is1)
    osem = (os0, os1)

    def start_in(v0, b):
        pltpu.async_copy(tabT.at[:, pl.ds(v0, _CV)], ibuf[b], isem[b])

    def wait_in(v0, b):
        pltpu.make_async_copy(tabT.at[:, pl.ds(v0, _CV)], ibuf[b],
                              isem[b]).wait()

    def start_out(v0, b):
        pltpu.async_copy(obuf[b], out_hbm.at[pl.ds(v0 * _D, _CV * _D)],
                         osem[b])

    def wait_out(v0, b):
        pltpu.make_async_copy(obuf[b],
                              out_hbm.at[pl.ds(v0 * _D, _CV * _D)],
                              osem[b]).wait()

    def compute(b):
        def grp(vg, carry):
            vvec = vg * 16 + iota
            fbase = vvec * _D
            for cb in range(_D // 16):
                for j in range(16):
                    cvec = cb * 16 + ((iota + j) & 15)
                    vals = plsc.load_gather(ibuf[b], [cvec, vvec])
                    plsc.store_scatter(obuf[b], [fbase + cvec], vals)
            return carry
        lax.fori_loop(0, _CV // 16, grp, 0)

    start_in(base, 0)
    start_in(base + _CV, 1)
    for k in (0, 1):
        b = k & 1
        wait_in(base + k * _CV, b)
        compute(b)
        start_out(base + k * _CV, b)
        start_in(base + (k + 2) * _CV, b)

    def steady(ko, carry):
        for b in range(2):
            v0 = base + (ko * 2 + b) * _CV
            wait_out(v0 - 2 * _CV, b)
            wait_in(v0, b)
            compute(b)
            start_out(v0, b)
            start_in(v0 + 2 * _CV, b)
        return carry

    lax.fori_loop(1, _CPW // 2 - 1, steady, 0)

    for k in (_CPW - 2, _CPW - 1):
        b = k & 1
        v0 = base + k * _CV
        wait_out(v0 - 2 * _CV, b)
        wait_in(v0, b)
        compute(b)
        start_out(v0, b)
    for k in (_CPW - 2, _CPW - 1):
        wait_out(base + k * _CV, k & 1)

    @pl.when(wid < 4)
    def _():
        v0 = _TAIL + wid * _CV
        pltpu.sync_copy(tabT.at[:, pl.ds(v0, _CV)], ibuf[0])
        compute(0)
        pltpu.sync_copy(obuf[0], out_hbm.at[pl.ds(v0 * _D, _CV * _D)])


def _emb_body(idxT, tab_hbm, tail_hbm, out_hbm, idx_v, tail_v,
              g0, g1, o0, o1, gs0, gs1, os0, os1):
    c = lax.axis_index("c")
    s = lax.axis_index("s")
    wid = s * 2 + c
    b0 = wid * _BPW
    iota = lax.iota(jnp.int32, 16)

    gbuf = (g0, g1)
    obuf = (o0, o1)
    gsem = (gs0, gs1)
    osem = (os0, os1)

    pltpu.sync_copy(idxT.at[:, pl.ds(b0, _BPW)], idx_v)
    pltpu.sync_copy(tail_hbm, tail_v)

    def start_gather(t, b):
        pltpu.async_copy(tab_hbm.at[idx_v.at[t]], gbuf[b], gsem[b])

    def wait_gather(t, b):
        pltpu.make_async_copy(tab_hbm.at[idx_v.at[t]], gbuf[b],
                              gsem[b]).wait()

    def start_out(t, b):
        pltpu.async_copy(obuf[b], out_hbm.at[t, :, wid, :, :], osem[b])

    def wait_out(t, b):
        pltpu.make_async_copy(obuf[b], out_hbm.at[t, :, wid, :, :],
                              osem[b]).wait()

    def compute(t, b):
        def grp(bg, carry):
            iv = idx_v[t, pl.ds(bg * 16, 16)]
            sc = jnp.where(iv != 0, jnp.float32(_SCALE), jnp.float32(0.0))
            blvec = bg * 16 + iota
            tmask = iv >= _VMAIN

            def sweep(patch):
                tridx = jnp.where(tmask, iv - _VMAIN, 0)
                for cb in range(_D // 16):
                    for j in range(16):
                        cvec = cb * 16 + ((iota + j) & 15)
                        vals = plsc.load_gather(gbuf[b], [blvec, cvec])
                        if patch:
                            tvals = plsc.load_gather(tail_v, [tridx, cvec])
                            vals = jnp.where(tmask, tvals, vals)
                        plsc.store_scatter(
                            obuf[b], [cvec >> 3, cvec & 7, blvec], vals * sc)

            @pl.when(jnp.logical_not(tmask.any()))
            def _():
                sweep(False)

            @pl.when(tmask.any())
            def _():
                sweep(True)
            return carry
        lax.fori_loop(0, _BPW // 16, grp, 0)

    start_gather(0, 0)
    start_gather(1, 1)
    for t in (0, 1):
        b = t & 1
        wait_gather(t, b)
        compute(t, b)
        start_out(t, b)
        start_gather(t + 2, b)

    def steady(to, carry):
        for b in range(2):
            t = to * 2 + b
            wait_out(t - 2, b)
            wait_gather(t, b)
            compute(t, b)
            start_out(t, b)
            start_gather(t + 2, b)
        return carry

    lax.fori_loop(1, _TOK // 2 - 1, steady, 0)

    for t in (_TOK - 2, _TOK - 1):
        b = t & 1
        wait_out(t - 2, b)
        wait_gather(t, b)
        compute(t, b)
        start_out(t, b)
    for t in (_TOK - 2, _TOK - 1):
        wait_out(t, t & 1)


def kernel(input, lookup_table):
    idxT = input.astype(jnp.int32).T
    tabT = lookup_table.T
    tab_tail = lookup_table[_VMAIN:, :]
    mesh = plsc.VectorSubcoreMesh(core_axis_name="c", subcore_axis_name="s")

    tab_flat = pl.kernel(
        _tr_body,
        out_type=jax.ShapeDtypeStruct((_VOC * _D,), jnp.float32),
        mesh=mesh,
        compiler_params=pltpu.CompilerParams(
            use_tc_tiling_on_sc=True, needs_layout_passes=False),
        scratch_types=[
            pltpu.VMEM((_D, _CV), jnp.float32),
            pltpu.VMEM((_D, _CV), jnp.float32),
            pltpu.VMEM((_CV * _D,), jnp.float32),
            pltpu.VMEM((_CV * _D,), jnp.float32),
            pltpu.SemaphoreType.DMA,
            pltpu.SemaphoreType.DMA,
            pltpu.SemaphoreType.DMA,
            pltpu.SemaphoreType.DMA,
        ],
    )(tabT)
    tab_lin = tab_flat.reshape(_VOC, _D)

    out5d = pl.kernel(
        _emb_body,
        out_type=jax.ShapeDtypeStruct((_TOK, _D // 8, _BATCH // 128, 8, 128),
                                      jnp.float32),
        mesh=mesh,
        compiler_params=pltpu.CompilerParams(
            use_tc_tiling_on_sc=False, needs_layout_passes=False),
        scratch_types=[
            pltpu.VMEM((_TOK, _BPW), jnp.int32),
            pltpu.VMEM((_D, _D), jnp.float32),
            pltpu.VMEM((_BPW, _D), jnp.float32),
            pltpu.VMEM((_BPW, _D), jnp.float32),
            pltpu.VMEM((_D // 8, 8, _BPW), jnp.float32),
            pltpu.VMEM((_D // 8, 8, _BPW), jnp.float32),
            pltpu.SemaphoreType.DMA,
            pltpu.SemaphoreType.DMA,
            pltpu.SemaphoreType.DMA,
            pltpu.SemaphoreType.DMA,
        ],
    )(idxT, tab_lin, tab_tail)

    return out5d.transpose(2, 4, 0, 1, 3).reshape(_BATCH, _TOK, _D)


# speedup vs baseline: 2.3844x; 1.1620x over previous
import jax
import jax.numpy as jnp
from jax import lax
from jax.experimental import pallas as pl
from jax.experimental.pallas import tpu as pltpu
from jax.experimental.pallas import tpu_sc as plsc

_VOC = 1000000
_VMAIN = 999936
_BATCH = 4096
_TOK = 200
_D = 64
_NW = 32
_BPW = _BATCH // _NW
_CV = 128
_CPW = _VMAIN // (_NW * _CV)
_STRIPE = _CPW * _CV
_TAIL = _NW * _STRIPE
_SCALE = 8.0


def _tr_body(tabT, out_hbm, i0, i1, o0, o1, is0, is1, os0, os1):
    c = lax.axis_index("c")
    s = lax.axis_index("s")
    wid = s * 2 + c
    base = wid * _STRIPE
    iota = lax.iota(jnp.int32, 16)

    ibuf = (i0, i1)
    obuf = (o0, o1)
    isem = (is0, is1)
    osem = (os0, os1)

    def start_in(v0, b):
        pltpu.async_copy(tabT.at[:, pl.ds(v0, _CV)], ibuf[b], isem[b])

    def wait_in(v0, b):
        pltpu.make_async_copy(tabT.at[:, pl.ds(v0, _CV)], ibuf[b],
                              isem[b]).wait()

    def start_out(v0, b):
        pltpu.async_copy(obuf[b], out_hbm.at[pl.ds(v0 * _D, _CV * _D)],
                         osem[b])

    def wait_out(v0, b):
        pltpu.make_async_copy(obuf[b],
                              out_hbm.at[pl.ds(v0 * _D, _CV * _D)],
                              osem[b]).wait()

    def compute(b):
        @plsc.parallel_loop(0, _CV // 16)
        def _(vg):
            vvec = vg * 16 + iota
            fbase = vvec * _D
            for cb in range(_D // 16):
                for j in range(16):
                    cvec = cb * 16 + ((iota + j) & 15)
                    vals = plsc.load_gather(ibuf[b], [cvec, vvec])
                    plsc.store_scatter(obuf[b], [fbase + cvec], vals)

    start_in(base, 0)
    start_in(base + _CV, 1)

    def steady(ko, carry):
        for b in range(2):
            k = ko * 2 + b
            v0 = base + k * _CV

            @pl.when(k >= 2)
            def _():
                wait_out(v0 - 2 * _CV, b)
            wait_in(v0, b)
            compute(b)
            start_out(v0, b)

            @pl.when(ko < _CPW // 2 - 1)
            def _():
                start_in(v0 + 2 * _CV, b)
        return carry

    lax.fori_loop(0, _CPW // 2, steady, 0)

    for k in (_CPW - 2, _CPW - 1):
        wait_out(base + k * _CV, k & 1)

    @pl.when(wid < 4)
    def _():
        v0 = _TAIL + wid * _CV
        pltpu.sync_copy(tabT.at[:, pl.ds(v0, _CV)], ibuf[0])
        compute(0)
        pltpu.sync_copy(obuf[0], out_hbm.at[pl.ds(v0 * _D, _CV * _D)])


def _emb_body(idxT, tab_hbm, tail_hbm, out_hbm, idx_v, tail_v,
              g0, g1, o0, o1, gs0, gs1, os0, os1):
    c = lax.axis_index("c")
    s = lax.axis_index("s")
    wid = s * 2 + c
    b0 = wid * _BPW
    iota = lax.iota(jnp.int32, 16)

    gbuf = (g0, g1)
    obuf = (o0, o1)
    gsem = (gs0, gs1)
    osem = (os0, os1)

    pltpu.sync_copy(idxT.at[:, pl.ds(b0, _BPW)], idx_v)
    pltpu.sync_copy(tail_hbm, tail_v)

    def start_gather(t, b):
        pltpu.async_copy(tab_hbm.at[idx_v.at[t]], gbuf[b], gsem[b])

    def wait_gather(t, b):
        pltpu.make_async_copy(tab_hbm.at[idx_v.at[t]], gbuf[b],
                              gsem[b]).wait()

    def start_out(t, b):
        pltpu.async_copy(obuf[b], out_hbm.at[t, :, wid, :, :], osem[b])

    def wait_out(t, b):
        pltpu.make_async_copy(obuf[b], out_hbm.at[t, :, wid, :, :],
                              osem[b]).wait()

    def compute(t, b):
        @plsc.parallel_loop(0, _BPW // 16)
        def _(bg):
            iv = idx_v[t, pl.ds(bg * 16, 16)]
            sc = jnp.where(iv != 0, jnp.float32(_SCALE), jnp.float32(0.0))
            blvec = bg * 16 + iota
            tmask = iv >= _VMAIN

            def sweep(patch):
                tridx = jnp.where(tmask, iv - _VMAIN, 0)
                for cb in range(_D // 16):
                    for j in range(16):
                        cvec = cb * 16 + ((iota + j) & 15)
                        vals = plsc.load_gather(gbuf[b], [blvec, cvec])
                        if patch:
                            tvals = plsc.load_gather(tail_v, [tridx, cvec])
                            vals = jnp.where(tmask, tvals, vals)
                        plsc.store_scatter(
                            obuf[b], [cvec >> 3, cvec & 7, blvec], vals * sc)

            @pl.when(jnp.logical_not(tmask.any()))
            def _():
                sweep(False)

            @pl.when(tmask.any())
            def _():
                sweep(True)

    start_gather(0, 0)
    start_gather(1, 1)

    def steady(to, carry):
        for b in range(2):
            t = to * 2 + b

            @pl.when(t >= 2)
            def _():
                wait_out(t - 2, b)
            wait_gather(t, b)
            compute(t, b)
            start_out(t, b)

            @pl.when(to < _TOK // 2 - 1)
            def _():
                start_gather(t + 2, b)
        return carry

    lax.fori_loop(0, _TOK // 2, steady, 0)

    for t in (_TOK - 2, _TOK - 1):
        wait_out(t, t & 1)


def kernel(input, lookup_table):
    idxT = input.astype(jnp.int32).T
    tabT = lookup_table.T
    tab_tail = lookup_table[_VMAIN:, :]
    mesh = plsc.VectorSubcoreMesh(core_axis_name="c", subcore_axis_name="s")

    tab_flat = pl.kernel(
        _tr_body,
        out_type=jax.ShapeDtypeStruct((_VOC * _D,), jnp.float32),
        mesh=mesh,
        compiler_params=pltpu.CompilerParams(
            use_tc_tiling_on_sc=True, needs_layout_passes=False),
        scratch_types=[
            pltpu.VMEM((_D, _CV), jnp.float32),
            pltpu.VMEM((_D, _CV), jnp.float32),
            pltpu.VMEM((_CV * _D,), jnp.float32),
            pltpu.VMEM((_CV * _D,), jnp.float32),
            pltpu.SemaphoreType.DMA,
            pltpu.SemaphoreType.DMA,
            pltpu.SemaphoreType.DMA,
            pltpu.SemaphoreType.DMA,
        ],
    )(tabT)
    tab_lin = tab_flat.reshape(_VOC, _D)

    out5d = pl.kernel(
        _emb_body,
        out_type=jax.ShapeDtypeStruct((_TOK, _D // 8, _BATCH // 128, 8, 128),
                                      jnp.float32),
        mesh=mesh,
        compiler_params=pltpu.CompilerParams(
            use_tc_tiling_on_sc=False, needs_layout_passes=False),
        scratch_types=[
            pltpu.VMEM((_TOK, _BPW), jnp.int32),
            pltpu.VMEM((_D, _D), jnp.float32),
            pltpu.VMEM((_BPW, _D), jnp.float32),
            pltpu.VMEM((_BPW, _D), jnp.float32),
            pltpu.VMEM((_D // 8, 8, _BPW), jnp.float32),
            pltpu.VMEM((_D // 8, 8, _BPW), jnp.float32),
            pltpu.SemaphoreType.DMA,
            pltpu.SemaphoreType.DMA,
            pltpu.SemaphoreType.DMA,
            pltpu.SemaphoreType.DMA,
        ],
    )(idxT, tab_lin, tab_tail)

    return out5d.transpose(2, 4, 0, 1, 3).reshape(_BATCH, _TOK, _D)
